# baseline (speedup 1.0000x reference)
import functools

import jax
import jax.numpy as jnp
from jax import lax
from jax.experimental import pallas as pl
from jax.experimental.pallas import tpu as pltpu
from jax.experimental.pallas import tpu_sc as plsc

NU = 50000
NI = 50000
N = NU + NI
D = 32
E = 1600000

NC = 2
NS = 16
LN = 128

E_ROWS = 12544
RT = E_ROWS // NS

SROWS = 16
NSUP = RT // SROWS
CE = 256
NCH = SROWS * LN // CE

HALF = N // NC
ACC_ROWS = 50048
ZPT = ACC_ROWS // NS


def _layer_body(x_hbm, src_hbm, dst_hbm, w_hbm, out_hbm, acc_sh,
                src_a, dst_a, w_a, src_b, dst_b, w_b,
                rows_a, rows_b, sem_sa, sem_sb, sem_ga, sem_gb):
    c = lax.axis_index("c")
    s = lax.axis_index("s")
    dst_base = c * HALF

    sbufs = ((src_a, dst_a, w_a, sem_sa), (src_b, dst_b, w_b, sem_sb))
    rbufs = ((rows_a, sem_ga), (rows_b, sem_gb))

    zeros16 = jnp.zeros((16,), jnp.float32)

    @plsc.parallel_loop(0, CE)
    def _zrow(i):
        rows_a[i, pl.ds(0, 16)] = zeros16
        rows_a[i, pl.ds(16, 16)] = zeros16

    zbase = s * ZPT
    for z in range(12):
        pltpu.sync_copy(rows_a, acc_sh.at[pl.ds(zbase + z * CE, CE)])
    pltpu.sync_copy(rows_a.at[pl.ds(0, ZPT - 12 * CE)],
                    acc_sh.at[pl.ds(zbase + 12 * CE, ZPT - 12 * CE)])
    plsc.subcore_barrier()

    def fire_super(u, sbuf):
        src_v, dst_v, w_v, sem = sbuf
        row0 = s * RT + u * SROWS
        pltpu.async_copy(src_hbm.at[pl.ds(row0, SROWS)], src_v, sem)
        pltpu.async_copy(dst_hbm.at[pl.ds(row0, SROWS)], dst_v, sem)
        pltpu.async_copy(w_hbm.at[pl.ds(row0 * LN, SROWS * LN)], w_v, sem)

    def process_super(u, sbuf):
        src_v, dst_v, w_v, sem = sbuf
        row0 = s * RT + u * SROWS
        pltpu.make_async_copy(src_hbm.at[pl.ds(row0, SROWS)], src_v, sem).wait()
        pltpu.make_async_copy(dst_hbm.at[pl.ds(row0, SROWS)], dst_v, sem).wait()
        pltpu.make_async_copy(w_hbm.at[pl.ds(row0 * LN, SROWS * LN)],
                              w_v, sem).wait()

        @plsc.parallel_loop(0, SROWS * LN // 16, unroll=2)
        def _loc(g2):
            j = g2 // (LN // 16)
            k = g2 - j * (LN // 16)
            dv = dst_v[j, pl.ds(k * 16, 16)]
            loc = dv - dst_base
            ok = (loc >= 0) & (loc < HALF)
            dst_v[j, pl.ds(k * 16, 16)] = jnp.where(ok, loc, HALF)

        def fire_g(ch, rbuf):
            rows_v, gsem = rbuf
            for j in range(2):
                pltpu.async_copy(x_hbm.at[src_v.at[2 * ch + j]],
                                 rows_v.at[pl.ds(j * LN, LN)], gsem)

        def proc_g(ch, rbuf):
            rows_v, gsem = rbuf
            for j in range(2):
                pltpu.make_async_copy(x_hbm.at[src_v.at[2 * ch + j]],
                                      rows_v.at[pl.ds(j * LN, LN)],
                                      gsem).wait()

            @plsc.parallel_loop(0, CE // 16, unroll=2)
            def _wmul(g2):
                w16 = w_v[pl.ds(ch * CE + g2 * 16, 16)]
                e0 = g2 * 16
                for i in range(16):
                    wv = jnp.take_along_axis(
                        w16, jnp.full((16,), i, jnp.int32), axis=0)
                    rows_v[e0 + i, pl.ds(0, 16)] = (
                        rows_v[e0 + i, pl.ds(0, 16)] * wv)
                    rows_v[e0 + i, pl.ds(16, 16)] = (
                        rows_v[e0 + i, pl.ds(16, 16)] * wv)

            for j in range(2):
                pltpu.sync_copy(rows_v.at[pl.ds(j * LN, LN)],
                                acc_sh.at[dst_v.at[2 * ch + j]], add=True)

        fire_g(0, rbufs[0])
        for p in range(NCH // 2):
            fire_g(2 * p + 1, rbufs[1])
            proc_g(2 * p, rbufs[0])
            if 2 * p + 2 < NCH:
                fire_g(2 * p + 2, rbufs[0])
            proc_g(2 * p + 1, rbufs[1])

    fire_super(0, sbufs[0])

    def _pair(q, _):
        @pl.when(2 * q + 1 < NSUP)
        def _():
            fire_super(2 * q + 1, sbufs[1])

        process_super(2 * q, sbufs[0])

        @pl.when(2 * q + 2 < NSUP)
        def _():
            fire_super(2 * q + 2, sbufs[0])

        @pl.when(2 * q + 1 < NSUP)
        def _():
            process_super(2 * q + 1, sbufs[1])

        return 0

    lax.fori_loop(0, (NSUP + 1) // 2, _pair, 0)
    plsc.subcore_barrier()

    blk0 = (s * 6250) // 16
    nb = ((s + 1) * 6250) // 16 - blk0
    wbase = blk0 * 8
    obase = dst_base + wbase
    for z in range(10):
        pltpu.sync_copy(acc_sh.at[pl.ds(wbase + z * 312, 312)],
                        out_hbm.at[pl.ds(obase + z * 312, 312)])

    @pl.when(nb == 391)
    def _():
        pltpu.sync_copy(acc_sh.at[pl.ds(wbase + 3120, 8)],
                        out_hbm.at[pl.ds(obase + 3120, 8)])


_layer = functools.partial(
    pl.kernel,
    out_type=jax.ShapeDtypeStruct((N, D), jnp.float32),
    mesh=plsc.VectorSubcoreMesh(core_axis_name="c", subcore_axis_name="s"),
    scratch_types=[
        pltpu.VMEM_SHARED((ACC_ROWS, D), jnp.float32),
        pltpu.VMEM((SROWS, LN), jnp.int32),
        pltpu.VMEM((SROWS, LN), jnp.int32),
        pltpu.VMEM((SROWS * LN,), jnp.float32),
        pltpu.VMEM((SROWS, LN), jnp.int32),
        pltpu.VMEM((SROWS, LN), jnp.int32),
        pltpu.VMEM((SROWS * LN,), jnp.float32),
        pltpu.VMEM((CE, D), jnp.float32),
        pltpu.VMEM((CE, D), jnp.float32),
        pltpu.SemaphoreType.DMA,
        pltpu.SemaphoreType.DMA,
        pltpu.SemaphoreType.DMA,
        pltpu.SemaphoreType.DMA,
    ],
    compiler_params=pltpu.CompilerParams(use_tc_tiling_on_sc=False),
)(_layer_body)


def _mean_body(a_ref, b_ref, c_ref, d_ref, o_ref):
    o_ref[...] = (a_ref[...] + b_ref[...] + c_ref[...] + d_ref[...]) * 0.25


def _mean4(x0, x1, x2, x3):
    rs = lambda x: x.reshape(25000, 128)
    spec = pl.BlockSpec((1000, 128), lambda i: (i, 0))
    out = pl.pallas_call(
        _mean_body,
        grid=(25,),
        in_specs=[spec] * 4,
        out_specs=spec,
        out_shape=jax.ShapeDtypeStruct((25000, 128), jnp.float32),
    )(rs(x0), rs(x1), rs(x2), rs(x3))
    return out.reshape(N, D)


def kernel(user_table, item_table, edge_index, edge_weight):
    x0 = jnp.concatenate([user_table, item_table], axis=0)
    pad = E_ROWS * LN - E
    src = jnp.concatenate([edge_index[0], jnp.zeros((pad,), jnp.int32)])
    dst = jnp.concatenate([edge_index[1], jnp.zeros((pad,), jnp.int32)])
    w = jnp.concatenate([edge_weight, jnp.zeros((pad,), jnp.float32)])
    src = src.reshape(E_ROWS, LN)
    dst = dst.reshape(E_ROWS, LN)

    x1 = _layer(x0, src, dst, w)
    x2 = _layer(x1, src, dst, w)
    x3 = _layer(x2, src, dst, w)
    out = _mean4(x0, x1, x2, x3)
    return out[:NU], out[NU:]
